# Initial kernel scaffold; baseline (speedup 1.0000x reference)
import jax
import jax.numpy as jnp
from jax.experimental import pallas as pl


def kernel(voxel_features, voxel_coords, batch_size, params):
    raise NotImplementedError("write your pallas kernel here")


# speedup vs baseline: 1.3152x; 1.3152x over previous
import functools

import jax
import jax.numpy as jnp
from jax.experimental import pallas as pl
import numpy as np

GRID_D, GRID_H, GRID_W = 16, 128, 128
EPS = 1e-5

OFFSETS = [(dz - 1, dy - 1, dx - 1) for dz in range(3) for dy in range(3) for dx in range(3)]


def _chunk_len(C, HW, W, budget=8 * 1024 * 1024):
    ch = HW
    while 27 * C * ch * 4 > budget and ch % 2 == 0 and ch > W:
        ch //= 2
    return ch


def _apply_val(xv, mv, mean, rs, g, b, res, *, mode, relu):
    if mode == 'identity':
        return xv
    xm = xv * mv
    y = (xm - mean[:, :1]) * rs[:, :1]
    y = y * g[:, :1] + b[:, :1]
    y = y * mv
    if res is not None:
        y = y + res
    if relu:
        y = jnp.maximum(y, 0.0)
    return y


def _conv_body(x_m1, x_0, x_p1, m_m1, m_0, m_p1, mean, rs, g, b,
               res_m1, res_0, res_p1, wf, out, slabp, col,
               *, C, Co, D, H, W, HW, CH, Wp, mode, relu, has_res, out_mask):
    z = pl.program_id(1)
    PADL = slabp.shape[-1]
    xs = (x_m1, x_0, x_p1)
    ms = (m_m1, m_0, m_p1)
    rsl = (res_m1, res_0, res_p1)
    for dz in range(3):
        valid = jnp.logical_and(z + (dz - 1) >= 0, z + (dz - 1) <= D - 1)
        vf = jnp.where(valid, 1.0, 0.0).astype(jnp.float32)
        xv = xs[dz][0, 0]
        mv = ms[dz][0, 0]
        res = rsl[dz][0, 0] if has_res else None
        val = _apply_val(xv, mv, mean, rs, g, b, res, mode=mode, relu=relu)
        slabp[dz, :, :Wp] = jnp.zeros((C, Wp), jnp.float32)
        slabp[dz, :, Wp:Wp + HW] = val * vf
        slabp[dz, :, Wp + HW:] = jnp.zeros((C, PADL - Wp - HW), jnp.float32)

    lane = jax.lax.broadcasted_iota(jnp.int32, (1, CH), 1)
    mask_lo = jnp.where(lane % W != 0, 1.0, 0.0)
    mask_hi = jnp.where(lane % W != W - 1, 1.0, 0.0)

    nch = HW // CH
    for c in range(nch):
        base = Wp + c * CH
        for k, (dz, dy, dx) in enumerate(OFFSETS):
            t = dy * W + dx
            sl = slabp[dz + 1, :, base + t:base + t + CH]
            if dx == -1:
                sl = sl * mask_lo
            elif dx == 1:
                sl = sl * mask_hi
            col[k * C:(k + 1) * C, :] = sl.astype(jnp.bfloat16)
        acc = jax.lax.dot_general(wf[...], col[...], (((1,), (0,)), ((), ())),
                                  preferred_element_type=jnp.float32)
        if out_mask:
            acc = acc * m_0[0, 0, :, c * CH:(c + 1) * CH]
        out[0, 0, :, c * CH:(c + 1) * CH] = acc


def conv_pallas(x, mask, wf, mean, rs, g, b, res, *, C, Co, D, H, W,
                mode='bn', relu=True, out_mask=False):
    B = x.shape[0]
    HW = H * W
    CH = _chunk_len(C, HW, W)
    Wp = W + 8
    PADL = Wp + HW + Wp
    has_res = res is not None
    if res is None:
        res = jnp.zeros((1, 1, 1, 1), jnp.float32)
    if mode == 'identity':
        mean = rs = g = b = jnp.zeros((C, 1), jnp.float32)

    def xmap(dz):
        return lambda bi, zi: (bi, jnp.clip(zi + dz, 0, D - 1), 0, 0)

    x_spec = lambda dz: pl.BlockSpec((1, 1, C, HW), xmap(dz))
    m_spec = lambda dz: pl.BlockSpec((1, 1, 1, HW), xmap(dz))
    if has_res:
        r_spec = lambda dz: pl.BlockSpec((1, 1, C, HW), xmap(dz))
    else:
        r_spec = lambda dz: pl.BlockSpec((1, 1, 1, 1), lambda bi, zi: (0, 0, 0, 0))
    v_spec = pl.BlockSpec((C, 1), lambda bi, zi: (0, 0))
    body = functools.partial(
        _conv_body, C=C, Co=Co, D=D, H=H, W=W, HW=HW, CH=CH, Wp=Wp,
        mode=mode, relu=relu, has_res=has_res, out_mask=out_mask)
    return pl.pallas_call(
        body,
        grid=(B, D),
        in_specs=[x_spec(-1), x_spec(0), x_spec(1),
                  m_spec(-1), m_spec(0), m_spec(1),
                  v_spec, v_spec, v_spec, v_spec,
                  r_spec(-1), r_spec(0), r_spec(1),
                  pl.BlockSpec((Co, 27 * C), lambda bi, zi: (0, 0))],
        out_specs=pl.BlockSpec((1, 1, Co, HW), lambda bi, zi: (bi, zi, 0, 0)),
        out_shape=jax.ShapeDtypeStruct((B, D, Co, HW), jnp.float32),
        scratch_shapes=[pltpu_vmem((3, C, PADL)), pltpu_vmem((27 * C, CH), jnp.bfloat16)],
    )(x, x, x, mask, mask, mask, mean, rs, g, b, res, res, res, wf)


def _apply_body(x_0, m_0, mean, rs, g, b, res_0, out, *, relu, has_res):
    res = res_0[0, 0] if has_res else None
    out[0, 0] = _apply_val(x_0[0, 0], m_0[0, 0], mean, rs, g, b, res,
                           mode='bn', relu=relu)


def apply_pallas(x, mask, mean, rs, g, b, res, *, C, D, HW, relu=True):
    B = x.shape[0]
    has_res = res is not None
    if res is None:
        res = jnp.zeros((1, 1, 1, 1), jnp.float32)
        r_spec = pl.BlockSpec((1, 1, 1, 1), lambda bi, zi: (0, 0, 0, 0))
    else:
        r_spec = pl.BlockSpec((1, 1, C, HW), lambda bi, zi: (bi, zi, 0, 0))
    v_spec = pl.BlockSpec((C, 1), lambda bi, zi: (0, 0))
    body = functools.partial(_apply_body, relu=relu, has_res=has_res)
    return pl.pallas_call(
        body,
        grid=(B, D),
        in_specs=[pl.BlockSpec((1, 1, C, HW), lambda bi, zi: (bi, zi, 0, 0)),
                  pl.BlockSpec((1, 1, 1, HW), lambda bi, zi: (bi, zi, 0, 0)),
                  v_spec, v_spec, v_spec, v_spec, r_spec],
        out_specs=pl.BlockSpec((1, 1, C, HW), lambda bi, zi: (bi, zi, 0, 0)),
        out_shape=jax.ShapeDtypeStruct((B, D, C, HW), jnp.float32),
    )(x, mask, mean, rs, g, b, res)


def _maxpool_body(m_m1, m_0, m_p1, out, slabp, *, D, H, W, HW, Wp):
    z = pl.program_id(1)
    PADL = slabp.shape[-1]
    ms = (m_m1, m_0, m_p1)
    for dz in range(3):
        valid = jnp.logical_and(z + (dz - 1) >= 0, z + (dz - 1) <= D - 1)
        vf = jnp.where(valid, 1.0, 0.0).astype(jnp.float32)
        slabp[dz, :1, :Wp] = jnp.zeros((1, Wp), jnp.float32)
        slabp[dz, :1, Wp:Wp + HW] = ms[dz][0, 0] * vf
        slabp[dz, :1, Wp + HW:] = jnp.zeros((1, PADL - Wp - HW), jnp.float32)
    lane = jax.lax.broadcasted_iota(jnp.int32, (1, HW), 1)
    mask_lo = jnp.where(lane % W != 0, 1.0, 0.0)
    mask_hi = jnp.where(lane % W != W - 1, 1.0, 0.0)
    acc = jnp.zeros((1, HW), jnp.float32)
    for (dz, dy, dx) in OFFSETS:
        t = dy * W + dx
        sl = slabp[dz + 1, :1, Wp + t:Wp + t + HW]
        if dx == -1:
            sl = sl * mask_lo
        elif dx == 1:
            sl = sl * mask_hi
        acc = jnp.maximum(acc, sl)
    out[0, 0] = acc


def maxpool_pallas(mask, *, D, H, W):
    B = mask.shape[0]
    HW = H * W
    Wp = W + 8
    PADL = Wp + HW + Wp
    body = functools.partial(_maxpool_body, D=D, H=H, W=W, HW=HW, Wp=Wp)

    def xmap(dz):
        return lambda bi, zi: (bi, jnp.clip(zi + dz, 0, D - 1), 0, 0)

    return pl.pallas_call(
        body,
        grid=(B, D),
        in_specs=[pl.BlockSpec((1, 1, 1, HW), xmap(-1)),
                  pl.BlockSpec((1, 1, 1, HW), xmap(0)),
                  pl.BlockSpec((1, 1, 1, HW), xmap(1))],
        out_specs=pl.BlockSpec((1, 1, 1, HW), lambda bi, zi: (bi, zi, 0, 0)),
        out_shape=jax.ShapeDtypeStruct((B, D, 1, HW), jnp.float32),
        scratch_shapes=[pltpu_vmem((3, 8, PADL))],
    )(mask, mask, mask)


def pltpu_vmem(shape, dtype=jnp.float32):
    from jax.experimental.pallas import tpu as pltpu
    return pltpu.VMEM(shape, dtype)


def _wf(w, cpad=None):
    Co, Ci = w.shape[0], w.shape[1]
    if cpad is not None and cpad != Ci:
        w = jnp.concatenate([w, jnp.zeros((Co, cpad - Ci, 3, 3, 3), w.dtype)], axis=1)
        Ci = cpad
    return jnp.transpose(w, (0, 2, 3, 4, 1)).reshape(Co, 27 * Ci).astype(jnp.bfloat16)


def _to_ncdhw(t, Dd, Hh, Ww):
    B, C = t.shape[0], t.shape[2]
    return jnp.transpose(t.reshape(B, Dd, C, Hh, Ww), (0, 2, 1, 3, 4))


def _bn_scalars(o_my, mask_my, n, Dd, Hh, Ww):
    o_nc = _to_ncdhw(o_my, Dd, Hh, Ww)
    m_nc = jnp.transpose(mask_my.reshape(o_my.shape[0], Dd, 1, Hh, Ww), (0, 2, 1, 3, 4))
    xm = o_nc * m_nc
    mean = xm.sum(axis=(0, 2, 3, 4)) / n
    var = (xm * xm).sum(axis=(0, 2, 3, 4)) / n - mean * mean
    rs = jax.lax.rsqrt(var + EPS)
    return mean[:, None], rs[:, None]


def _stride_slice(x, D, H, W, C):
    B = x.shape[0]
    x = x.reshape(B, D, C, H, W)[:, ::2, :, ::2, ::2]
    D2, H2, W2 = x.shape[1], x.shape[3], x.shape[4]
    return x.reshape(B, D2, C, H2 * W2), D2, H2, W2


def _zero_stuff(x, D, H, W, C, Dt, Ht, Wt):
    B = x.shape[0]
    u = jnp.zeros((B, Dt, C, Ht, Wt), jnp.float32)
    u = u.at[:, 0:2 * D:2, :, 0:2 * H:2, 0:2 * W:2].set(x.reshape(B, D, C, H, W))
    return u.reshape(B, Dt, C, Ht * Wt)


def kernel(voxel_features, voxel_coords, batch_size, params):
    B = 2
    D, H, W = GRID_D, GRID_H, GRID_W
    HW = H * W
    Cin = voxel_features.shape[1]
    b = voxel_coords[:, 0] % batch_size
    z = voxel_coords[:, 1]
    y = voxel_coords[:, 2]
    xw = voxel_coords[:, 3]
    dense = jnp.zeros((B, D, H, W, Cin), jnp.float32).at[b, z, y, xw].set(voxel_features)
    occ = jnp.zeros((B, D, H, W), jnp.float32).at[b, z, y, xw].set(1.0)
    dense_nc = jnp.transpose(dense, (0, 4, 1, 2, 3))
    m0 = occ.reshape(B, D, 1, HW)
    n0 = jnp.maximum(m0.sum(), 1.0)

    dummy = None

    def vcol(v):
        return v[:, None]

    def stage_blocks(o_raw, mean, rs, g, bb, res_prime, mask, n, blocks, C, Dd, Hh, Ww):
        HWl = Hh * Ww
        cur = (o_raw, mean, rs, g, bb, res_prime)
        for p in blocks:
            o_prev, mn, rv, gv, bv, resv = cur
            if mn is None:
                blk_in = o_prev
                o1 = conv_pallas(o_prev, mask, _wf(p['w1']), None, None, None, None, None,
                                 C=C, Co=C, D=Dd, H=Hh, W=Ww, mode='identity')
            else:
                blk_in = apply_pallas(o_prev, mask, mn, rv, gv, bv, resv,
                                      C=C, D=Dd, HW=HWl, relu=True)
                o1 = conv_pallas(o_prev, mask, _wf(p['w1']), mn, rv, gv, bv, resv,
                                 C=C, Co=C, D=Dd, H=Hh, W=Ww)
            mn1, rs1 = _bn_scalars(o1, mask, n, Dd, Hh, Ww)
            o2 = conv_pallas(o1, mask, _wf(p['w2']), mn1, rs1,
                             vcol(p['g1']), vcol(p['b1']), None,
                             C=C, Co=C, D=Dd, H=Hh, W=Ww)
            mn2, rs2 = _bn_scalars(o2, mask, n, Dd, Hh, Ww)
            cur = (o2, mn2, rs2, vcol(p['g2']), vcol(p['b2']), blk_in)
        return cur

    def downsample(cur, mask, pd, C, Co, Dd, Hh, Ww):
        o_prev, mn, rv, gv, bv, resv = cur
        o_full = conv_pallas(o_prev, mask, _wf(pd['w']), mn, rv, gv, bv, resv,
                             C=C, Co=Co, D=Dd, H=Hh, W=Ww)
        o_s, D2, H2, W2 = _stride_slice(o_full, Dd, Hh, Ww, Co)
        mfull = maxpool_pallas(mask, D=Dd, H=Hh, W=Ww)
        m_s, _, _, _ = _stride_slice(mfull, Dd, Hh, Ww, 1)
        n2 = jnp.maximum(m_s.sum(), 1.0)
        mn2, rs2 = _bn_scalars(o_s, m_s, n2, D2, H2, W2)
        return (o_s, mn2, rs2, vcol(pd['g']), vcol(pd['b']), None), m_s, n2, D2, H2, W2

    p = params
    dense_my = jnp.transpose(dense_nc, (0, 2, 1, 3, 4)).reshape(B, D, Cin, HW)
    o0 = conv_pallas(dense_my, m0, _wf(p['w_in']), None, None, None, None, None,
                     C=Cin, Co=16, D=D, H=H, W=W, mode='identity')
    mn0, rs0 = _bn_scalars(o0, m0, n0, D, H, W)
    cur1 = (o0, mn0, rs0, vcol(p['g_in']), vcol(p['b_in']), None)

    cur1 = stage_blocks(*cur1[:6], m0, n0, p['c1'], 16, D, H, W)
    cur2, m2, n2, D2, H2, W2 = downsample(cur1, m0, p['d1'], 16, 32, D, H, W)
    cur2 = stage_blocks(*cur2[:6], m2, n2, p['c2'], 32, D2, H2, W2)
    cur3, m3, n3, D3, H3, W3 = downsample(cur2, m2, p['d2'], 32, 64, D2, H2, W2)
    cur3 = stage_blocks(*cur3[:6], m3, n3, p['c3'], 64, D3, H3, W3)
    cur4, m4, n4, D4, H4, W4 = downsample(cur3, m3, p['d3'], 64, 128, D3, H3, W3)
    cur4 = stage_blocks(*cur4[:6], m4, n4, p['c4'], 128, D4, H4, W4)
    cur5, m5, n5, D5, H5, W5 = downsample(cur4, m4, p['d4'], 128, 128, D4, H4, W4)
    cur5 = stage_blocks(*cur5[:6], m5, n5, p['c5'], 128, D5, H5, W5)
    cur6, m6, n6, D6, H6, W6 = downsample(cur5, m5, p['d5'], 128, 128, D5, H5, W5)
    cur6 = stage_blocks(*cur6[:6], m6, n6, p['c6'], 128, D6, H6, W6)

    def finalize(cur, mask, Dd, Hh, Ww):
        o_prev, mn, rv, gv, bv, resv = cur
        return apply_pallas(o_prev, mask, mn, rv, gv, bv, resv,
                            C=128, D=Dd, HW=Hh * Ww, relu=True)

    x4 = finalize(cur4, m4, D4, H4, W4)
    x5 = finalize(cur5, m5, D5, H5, W5)
    x6 = finalize(cur6, m6, D6, H6, W6)

    u5 = _zero_stuff(x5, D5, H5, W5, 128, D4, H4, W4)
    x5_up = conv_pallas(u5, m4, _wf(p['up5']), dummy, dummy, dummy, dummy, None,
                        C=128, Co=128, D=D4, H=H4, W=W4, mode='identity', out_mask=True)
    u6 = _zero_stuff(x6, D6, H6, W6, 128, D5, H5, W5)
    t6 = conv_pallas(u6, m5, _wf(p['up6a']), dummy, dummy, dummy, dummy, None,
                     C=128, Co=128, D=D5, H=H5, W=W5, mode='identity', out_mask=True)
    u6b = _zero_stuff(t6, D5, H5, W5, 128, D4, H4, W4)
    x6_up = conv_pallas(u6b, m4, _wf(p['up6b']), dummy, dummy, dummy, dummy, None,
                        C=128, Co=128, D=D4, H=H4, W=W4, mode='identity', out_mask=True)

    out = jnp.concatenate([_to_ncdhw(x4, D4, H4, W4),
                           _to_ncdhw(x5_up, D4, H4, W4),
                           _to_ncdhw(x6_up, D4, H4, W4)], axis=1)
    return out
